# baseline (speedup 1.0000x reference)
import functools

import jax
import jax.numpy as jnp
from jax.experimental import pallas as pl
from jax.experimental.pallas import tpu as pltpu

INPUT_DIM = 768
N_EXPERTS = 8
EXPERT_DIM = 128
HID = N_EXPERTS * EXPERT_DIM
TOP_K = 2
TILE_B = 1024


def _moe_kernel(x_ref, wg_ref, w1_ref, b1_ref, w2_ref, b2_ref,
                out_ref, gw_ref):
    xt = x_ref[...]

    logits = jax.lax.dot_general(
        xt, wg_ref[...], (((1,), (1,)), ((), ())),
        preferred_element_type=jnp.float32)
    ids = jax.lax.broadcasted_iota(jnp.int32, logits.shape, 1)
    m1 = jnp.max(logits, axis=-1, keepdims=True)
    i1 = jnp.min(jnp.where(logits == m1, ids, N_EXPERTS),
                 axis=-1, keepdims=True)
    masked = jnp.where(ids == i1, -jnp.inf, logits)
    m2 = jnp.max(masked, axis=-1, keepdims=True)
    i2 = jnp.min(jnp.where(masked == m2, ids, N_EXPERTS),
                 axis=-1, keepdims=True)
    e2 = jnp.exp(m2 - m1)
    denom = 1.0 + e2
    p1 = 1.0 / denom
    p2 = e2 / denom
    gw = (jnp.where(ids == i1, p1, 0.0) +
          jnp.where(ids == i2, p2, 0.0))
    gw_ref[...] = gw

    h = jax.lax.dot_general(
        xt.astype(jnp.bfloat16), w1_ref[...], (((1,), (1,)), ((), ())),
        preferred_element_type=jnp.float32) + b1_ref[...]
    h = 0.5 * h * (1.0 + jax.lax.erf(h * 0.7071067811865476))

    hs = jnp.concatenate(
        [h[:, e * EXPERT_DIM:(e + 1) * EXPERT_DIM] * gw[:, e:e + 1]
         for e in range(N_EXPERTS)], axis=1)
    out = jax.lax.dot_general(
        hs.astype(jnp.bfloat16), w2_ref[...], (((1,), (0,)), ((), ())),
        preferred_element_type=jnp.float32)
    out = out + jax.lax.dot_general(
        gw, b2_ref[...], (((1,), (0,)), ((), ())),
        preferred_element_type=jnp.float32)
    out_ref[...] = out


@functools.partial(jax.jit, static_argnames=())
def kernel(x, Wg, W1, b1, W2, b2):
    B = x.shape[0]
    grid = (B // TILE_B,)
    w1f = W1.reshape(HID, INPUT_DIM)
    w2f = W2.transpose(0, 2, 1).reshape(HID, EXPERT_DIM)
    b1f = b1.reshape(1, HID)
    full = lambda *shape: pl.BlockSpec(shape, lambda i: (0,) * len(shape))
    out, gw = pl.pallas_call(
        _moe_kernel,
        grid=grid,
        in_specs=[
            pl.BlockSpec((TILE_B, INPUT_DIM), lambda i: (i, 0)),
            full(N_EXPERTS, INPUT_DIM),
            full(HID, INPUT_DIM),
            full(1, HID),
            full(HID, EXPERT_DIM),
            full(N_EXPERTS, EXPERT_DIM),
        ],
        out_specs=[
            pl.BlockSpec((TILE_B, EXPERT_DIM), lambda i: (i, 0)),
            pl.BlockSpec((TILE_B, N_EXPERTS), lambda i: (i, 0)),
        ],
        out_shape=[
            jax.ShapeDtypeStruct((B, EXPERT_DIM), jnp.float32),
            jax.ShapeDtypeStruct((B, N_EXPERTS), jnp.float32),
        ],
    )(x, Wg, w1f.astype(jnp.bfloat16), b1f, w2f.astype(jnp.bfloat16), b2)
    return out, gw


# speedup vs baseline: 1.2703x; 1.2703x over previous
import functools

import jax
import jax.numpy as jnp
from jax.experimental import pallas as pl

INPUT_DIM = 768
N_EXPERTS = 8
EXPERT_DIM = 128
HID = N_EXPERTS * EXPERT_DIM
HALF = HID // 2
TOP_K = 2
TILE_B = 512


def _gating(logits):
    ids = jax.lax.broadcasted_iota(jnp.int32, logits.shape, 1)
    m1 = jnp.max(logits, axis=-1, keepdims=True)
    i1 = jnp.min(jnp.where(logits == m1, ids, N_EXPERTS),
                 axis=-1, keepdims=True)
    masked = jnp.where(ids == i1, -jnp.inf, logits)
    m2 = jnp.max(masked, axis=-1, keepdims=True)
    i2 = jnp.min(jnp.where(masked == m2, ids, N_EXPERTS),
                 axis=-1, keepdims=True)
    e2 = jnp.exp(m2 - m1)
    denom = 1.0 + e2
    return (jnp.where(ids == i1, 1.0 / denom, 0.0) +
            jnp.where(ids == i2, e2 / denom, 0.0))


def _half(xt, wg, w1a, w1b, b1, w2, b2):
    logits = jax.lax.dot_general(
        xt, wg, (((1,), (1,)), ((), ())),
        preferred_element_type=jnp.float32)
    gw = _gating(logits)

    ha = jax.lax.dot_general(
        xt, w1a, (((1,), (1,)), ((), ())),
        preferred_element_type=jnp.float32)
    hb = jax.lax.dot_general(
        xt, w1b, (((1,), (1,)), ((), ())),
        preferred_element_type=jnp.float32)
    h = jnp.concatenate([ha, hb], axis=1) + b1
    h = 0.5 * h * (1.0 + jax.lax.erf(h * 0.7071067811865476))

    hs = jnp.concatenate(
        [h[:, e * EXPERT_DIM:(e + 1) * EXPERT_DIM] * gw[:, e:e + 1]
         for e in range(N_EXPERTS)], axis=1)
    out = jax.lax.dot_general(
        hs, w2, (((1,), (0,)), ((), ())),
        preferred_element_type=jnp.float32)
    out = out + jax.lax.dot_general(
        gw, b2, (((1,), (0,)), ((), ())),
        preferred_element_type=jnp.float32)
    return out, gw


def _moe_kernel(xa_ref, xb_ref, wg_ref, w1a_ref, w1b_ref, b1_ref,
                w2_ref, b2_ref, out_ref, gw_ref):
    wg = wg_ref[...]
    w1a = w1a_ref[...]
    w1b = w1b_ref[...]
    b1 = b1_ref[...]
    w2 = w2_ref[...]
    b2 = b2_ref[...]
    out_a, gw_a = _half(xa_ref[...], wg, w1a, w1b, b1, w2, b2)
    out_b, gw_b = _half(xb_ref[...], wg, w1a, w1b, b1, w2, b2)
    out_ref[:TILE_B, :] = out_a
    out_ref[TILE_B:, :] = out_b
    gw_ref[:TILE_B, :] = gw_a
    gw_ref[TILE_B:, :] = gw_b


@functools.partial(jax.jit, static_argnames=())
def kernel(x, Wg, W1, b1, W2, b2):
    B = x.shape[0]
    grid = (B // (2 * TILE_B),)
    w1f = W1.reshape(HID, INPUT_DIM)
    w2f = W2.transpose(0, 2, 1).reshape(HID, EXPERT_DIM)
    b1f = b1.reshape(1, HID)
    full = lambda *shape: pl.BlockSpec(shape, lambda i: (0,) * len(shape))
    out, gw = pl.pallas_call(
        _moe_kernel,
        grid=grid,
        in_specs=[
            pl.BlockSpec((TILE_B, INPUT_DIM), lambda i: (2 * i, 0)),
            pl.BlockSpec((TILE_B, INPUT_DIM), lambda i: (2 * i + 1, 0)),
            full(N_EXPERTS, INPUT_DIM),
            pl.BlockSpec((HALF, INPUT_DIM), lambda i: (0, 0)),
            pl.BlockSpec((HALF, INPUT_DIM), lambda i: (1, 0)),
            full(1, HID),
            full(HID, EXPERT_DIM),
            full(N_EXPERTS, EXPERT_DIM),
        ],
        out_specs=[
            pl.BlockSpec((2 * TILE_B, EXPERT_DIM), lambda i: (i, 0)),
            pl.BlockSpec((2 * TILE_B, N_EXPERTS), lambda i: (i, 0)),
        ],
        out_shape=[
            jax.ShapeDtypeStruct((B, EXPERT_DIM), jnp.float32),
            jax.ShapeDtypeStruct((B, N_EXPERTS), jnp.float32),
        ],
    )(x, x, Wg, w1f, w1f, b1f, w2f, b2)
    return out, gw
